# Initial kernel scaffold; baseline (speedup 1.0000x reference)
import jax
import jax.numpy as jnp
from jax.experimental import pallas as pl


def kernel(x, edge_index, batch, W1, b1, W2, b2, Wf, bf):
    raise NotImplementedError("write your pallas kernel here")


# speedup vs baseline: 22.1346x; 22.1346x over previous
import functools

import jax
import jax.numpy as jnp
from jax import lax
from jax.experimental import pallas as pl
from jax.experimental.pallas import tpu as pltpu
from jax.experimental.pallas import tpu_sc as plsc

N = 100000
NPAD = 102400
E = 3200000
G = 128
NC, NS = 2, 16
NW = NC * NS
ROWS_PER_TEC = 784
EPW = ROWS_PER_TEC * 128
ETOT = EPW * NW
CHUNK_ROWS = 4
N_CHUNKS = ROWS_PER_TEC // CHUNK_ROWS
ROWS_PER_TEC2 = ETOT // 128 // NS
N_CHUNKS2 = ROWS_PER_TEC2 // CHUNK_ROWS

_MESH = dict(core_axis_name="c", subcore_axis_name="s", num_cores=NC,
             num_subcores=NS)


def _p1_deg(dst2d, ones1024, zeros6400):
    @functools.partial(
        pl.kernel,
        out_type=jax.ShapeDtypeStruct((NC, NPAD, 16), jnp.float32),
        mesh=plsc.VectorSubcoreMesh(**_MESH),
        compiler_params=pltpu.CompilerParams(use_tc_tiling_on_sc=False),
        scratch_types=[
            pltpu.VMEM((CHUNK_ROWS, 128), jnp.int32),
            pltpu.VMEM((128, 16), jnp.float32),
            pltpu.VMEM_SHARED((NPAD, 16), jnp.float32),
            pltpu.SemaphoreType.DMA,
        ],
    )
    def k(dst_hbm, ones_hbm, z_hbm, out_hbm, didx, obuf, acc, sem):
        c = lax.axis_index("c")
        s = lax.axis_index("s")
        t = c * NS + s
        pltpu.sync_copy(z_hbm, acc.at[pl.ds(s * 6400, 6400)])
        pltpu.sync_copy(ones_hbm.at[pl.ds(0, 128)], obuf)
        plsc.subcore_barrier()
        base = t * ROWS_PER_TEC

        def chunk(j, carry):
            pltpu.async_copy(
                dst_hbm.at[pl.ds(base + j * CHUNK_ROWS, CHUNK_ROWS)],
                didx, sem).wait()
            for r in range(CHUNK_ROWS):
                pltpu.sync_copy(obuf, acc.at[didx.at[r]], add=True)
            return carry

        lax.fori_loop(0, N_CHUNKS, chunk, 0)
        plsc.subcore_barrier()
        pltpu.sync_copy(acc.at[pl.ds(s * 6400, 6400)],
                        out_hbm.at[c, pl.ds(s * 6400, 6400)])

    return k(dst2d, ones1024, zeros6400)


def _p2_agg1(src2d, dst2d, g1, zeros6400):
    @functools.partial(
        pl.kernel,
        out_type=jax.ShapeDtypeStruct((NC, NPAD, 16), jnp.float32),
        mesh=plsc.VectorSubcoreMesh(**_MESH),
        compiler_params=pltpu.CompilerParams(use_tc_tiling_on_sc=False),
        scratch_types=[
            pltpu.VMEM((CHUNK_ROWS, 128), jnp.int32),
            pltpu.VMEM((CHUNK_ROWS, 128), jnp.int32),
            pltpu.VMEM((CHUNK_ROWS * 128, 16), jnp.float32),
            pltpu.VMEM_SHARED((NPAD, 16), jnp.float32),
            pltpu.SemaphoreType.DMA,
        ],
    )
    def k(src_hbm, dst_hbm, g_hbm, z_hbm, out_hbm, sidx, didx, gbuf, acc, sem):
        c = lax.axis_index("c")
        s = lax.axis_index("s")
        t = c * NS + s
        pltpu.sync_copy(z_hbm, acc.at[pl.ds(s * 6400, 6400)])
        plsc.subcore_barrier()
        base = t * ROWS_PER_TEC

        def chunk(j, carry):
            row0 = base + j * CHUNK_ROWS
            pltpu.async_copy(src_hbm.at[pl.ds(row0, CHUNK_ROWS)], sidx,
                             sem).wait()
            pltpu.async_copy(dst_hbm.at[pl.ds(row0, CHUNK_ROWS)], didx,
                             sem).wait()
            for r in range(CHUNK_ROWS):
                pltpu.async_copy(g_hbm.at[sidx.at[r]],
                                 gbuf.at[pl.ds(r * 128, 128)], sem).wait()
            for r in range(CHUNK_ROWS):
                pltpu.sync_copy(gbuf.at[pl.ds(r * 128, 128)],
                                acc.at[didx.at[r]], add=True)
            return carry

        lax.fori_loop(0, N_CHUNKS, chunk, 0)
        plsc.subcore_barrier()
        pltpu.sync_copy(acc.at[pl.ds(s * 6400, 6400)],
                        out_hbm.at[c, pl.ds(s * 6400, 6400)])

    return k(src2d, dst2d, g1, zeros6400)


def _p3_agg2(src2d, dst2d, g2a, g2b, zeros6400):
    @functools.partial(
        pl.kernel,
        out_type=jax.ShapeDtypeStruct((NC, NPAD, 16), jnp.float32),
        mesh=plsc.VectorSubcoreMesh(**_MESH),
        compiler_params=pltpu.CompilerParams(use_tc_tiling_on_sc=False),
        scratch_types=[
            pltpu.VMEM((CHUNK_ROWS, 128), jnp.int32),
            pltpu.VMEM((CHUNK_ROWS, 128), jnp.int32),
            pltpu.VMEM((CHUNK_ROWS * 128, 16), jnp.float32),
            pltpu.VMEM_SHARED((NPAD, 16), jnp.float32),
            pltpu.SemaphoreType.DMA,
        ],
    )
    def k(src_hbm, dst_hbm, ga_hbm, gb_hbm, z_hbm, out_hbm,
          sidx, didx, gbuf, acc, sem):
        c = lax.axis_index("c")
        s = lax.axis_index("s")
        pltpu.sync_copy(z_hbm, acc.at[pl.ds(s * 6400, 6400)])
        plsc.subcore_barrier()
        base = s * ROWS_PER_TEC2

        def run(g_hbm):
            def chunk(j, carry):
                row0 = base + j * CHUNK_ROWS
                pltpu.async_copy(src_hbm.at[pl.ds(row0, CHUNK_ROWS)], sidx,
                                 sem).wait()
                pltpu.async_copy(dst_hbm.at[pl.ds(row0, CHUNK_ROWS)], didx,
                                 sem).wait()
                for r in range(CHUNK_ROWS):
                    pltpu.async_copy(g_hbm.at[sidx.at[r]],
                                     gbuf.at[pl.ds(r * 128, 128)], sem).wait()
                for r in range(CHUNK_ROWS):
                    pltpu.sync_copy(gbuf.at[pl.ds(r * 128, 128)],
                                    acc.at[didx.at[r]], add=True)
                return carry
            lax.fori_loop(0, N_CHUNKS2, chunk, 0)

        @pl.when(c == 0)
        def _():
            run(ga_hbm)

        @pl.when(c == 1)
        def _():
            run(gb_hbm)

        plsc.subcore_barrier()
        pltpu.sync_copy(acc.at[pl.ds(s * 6400, 6400)],
                        out_hbm.at[c, pl.ds(s * 6400, 6400)])

    return k(src2d, dst2d, g2a, g2b, zeros6400)


def _p4_pool(p, batch2d, z32, zeros6400, ones128):
    @functools.partial(
        pl.kernel,
        out_type=(jax.ShapeDtypeStruct((NC, 256, 32), jnp.float32),
                  jax.ShapeDtypeStruct((NC, 256, 16), jnp.float32)),
        mesh=plsc.VectorSubcoreMesh(**_MESH),
        compiler_params=pltpu.CompilerParams(use_tc_tiling_on_sc=False),
        scratch_types=[
            pltpu.VMEM((128, 32), jnp.float32),
            pltpu.VMEM((25, 128), jnp.int32),
            pltpu.VMEM((128, 16), jnp.float32),
            pltpu.VMEM_SHARED((256, 32), jnp.float32),
            pltpu.VMEM_SHARED((256, 16), jnp.float32),
        ],
    )
    def k(p_hbm, b_hbm, z32_hbm, z16_hbm, ones_hbm, outs_hbm, outc_hbm,
          pbuf, bidx, onesv, pacc, cacc):
        c = lax.axis_index("c")
        s = lax.axis_index("s")
        t = c * NS + s

        @pl.when(s == 0)
        def _():
            pltpu.sync_copy(z32_hbm, pacc)
            pltpu.sync_copy(z16_hbm.at[pl.ds(0, 256)], cacc)

        pltpu.sync_copy(ones_hbm.at[pl.ds(0, 128)], onesv)
        pltpu.sync_copy(b_hbm.at[pl.ds(t * 25, 25)], bidx)
        plsc.subcore_barrier()
        for r in range(25):
            pltpu.sync_copy(p_hbm.at[pl.ds(t * 3200 + r * 128, 128)], pbuf)
            pltpu.sync_copy(pbuf, pacc.at[bidx.at[r]], add=True)
            pltpu.sync_copy(onesv, cacc.at[bidx.at[r]], add=True)
        plsc.subcore_barrier()

        @pl.when(s == 0)
        def _():
            pltpu.sync_copy(pacc, outs_hbm.at[c])
            pltpu.sync_copy(cacc, outc_hbm.at[c])

    return k(p, batch2d, z32, zeros6400, ones128)


def _t1(xp, W1p, d0, d1):
    def body(x_ref, w_ref, d0_ref, d1_ref, dinv_ref, g1_ref):
        deg = d0_ref[...][:, 0:1] + d1_ref[...][:, 0:1] + 1.0
        dinv = lax.rsqrt(deg)
        h = jnp.dot(x_ref[...], w_ref[...], preferred_element_type=jnp.float32)
        dinv_ref[...] = dinv
        g1_ref[...] = h * dinv

    return pl.pallas_call(
        body,
        grid=(NPAD // 2048,),
        in_specs=[
            pl.BlockSpec((2048, 8), lambda j: (j, 0)),
            pl.BlockSpec((8, 16), lambda j: (0, 0)),
            pl.BlockSpec((2048, 16), lambda j: (j, 0)),
            pl.BlockSpec((2048, 16), lambda j: (j, 0)),
        ],
        out_specs=[
            pl.BlockSpec((2048, 1), lambda j: (j, 0)),
            pl.BlockSpec((2048, 16), lambda j: (j, 0)),
        ],
        out_shape=[
            jax.ShapeDtypeStruct((NPAD, 1), jnp.float32),
            jax.ShapeDtypeStruct((NPAD, 16), jnp.float32),
        ],
    )(xp, W1p, d0, d1)


def _t2(a0, a1, g1, dinv, b1r, W2):
    def body(a0_ref, a1_ref, g1_ref, d_ref, b_ref, w_ref, ga_ref, gb_ref):
        dinv = d_ref[...]
        o1 = jnp.maximum((a0_ref[...] + a1_ref[...] + g1_ref[...]) * dinv
                         + b_ref[...], 0.0)
        h2 = jnp.dot(o1, w_ref[...], preferred_element_type=jnp.float32)
        g2 = h2 * dinv
        ga_ref[...] = g2[:, :16]
        gb_ref[...] = g2[:, 16:]

    return pl.pallas_call(
        body,
        grid=(NPAD // 2048,),
        in_specs=[
            pl.BlockSpec((2048, 16), lambda j: (j, 0)),
            pl.BlockSpec((2048, 16), lambda j: (j, 0)),
            pl.BlockSpec((2048, 16), lambda j: (j, 0)),
            pl.BlockSpec((2048, 1), lambda j: (j, 0)),
            pl.BlockSpec((1, 16), lambda j: (0, 0)),
            pl.BlockSpec((16, 32), lambda j: (0, 0)),
        ],
        out_specs=[
            pl.BlockSpec((2048, 16), lambda j: (j, 0)),
            pl.BlockSpec((2048, 16), lambda j: (j, 0)),
        ],
        out_shape=[
            jax.ShapeDtypeStruct((NPAD, 16), jnp.float32),
            jax.ShapeDtypeStruct((NPAD, 16), jnp.float32),
        ],
    )(a0, a1, g1, dinv, b1r, W2)


def _t3(a2a, a2b, g2a, g2b, dinv, b2r):
    def body(aa_ref, ab_ref, ga_ref, gb_ref, d_ref, b_ref, p_ref):
        dinv = d_ref[...]
        b = b_ref[...]
        oa = jnp.maximum((aa_ref[...] + ga_ref[...]) * dinv + b[:, :16], 0.0)
        ob = jnp.maximum((ab_ref[...] + gb_ref[...]) * dinv + b[:, 16:], 0.0)
        p_ref[...] = jnp.concatenate([oa, ob], axis=1)

    return pl.pallas_call(
        body,
        grid=(NPAD // 2048,),
        in_specs=[
            pl.BlockSpec((2048, 16), lambda j: (j, 0)),
            pl.BlockSpec((2048, 16), lambda j: (j, 0)),
            pl.BlockSpec((2048, 16), lambda j: (j, 0)),
            pl.BlockSpec((2048, 16), lambda j: (j, 0)),
            pl.BlockSpec((2048, 1), lambda j: (j, 0)),
            pl.BlockSpec((1, 32), lambda j: (0, 0)),
        ],
        out_specs=pl.BlockSpec((2048, 32), lambda j: (j, 0)),
        out_shape=jax.ShapeDtypeStruct((NPAD, 32), jnp.float32),
    )(a2a, a2b, g2a, g2b, dinv, b2r)


def _t4(ps, cs, Wf, bfr):
    def body(ps_ref, cs_ref, wf_ref, bf_ref, out_ref):
        ssum = ps_ref[0, :G, :] + ps_ref[1, :G, :]
        cnt = cs_ref[0, :G, 0:1] + cs_ref[1, :G, 0:1]
        pooled = ssum / jnp.maximum(cnt, 1.0)
        out_ref[...] = jnp.dot(pooled, wf_ref[...],
                               preferred_element_type=jnp.float32) + bf_ref[...]

    return pl.pallas_call(
        body,
        out_shape=jax.ShapeDtypeStruct((G, 3), jnp.float32),
    )(ps, cs, Wf, bfr)


def kernel(x, edge_index, batch, W1, b1, W2, b2, Wf, bf):
    f32 = jnp.float32
    xp = jnp.zeros((NPAD, 8), f32).at[:N, :5].set(x)
    W1p = jnp.zeros((8, 16), f32).at[:5, :].set(W1)
    pad_e = ETOT - E
    src2d = jnp.concatenate(
        [edge_index[0], jnp.full((pad_e,), NPAD - 1, jnp.int32)]
    ).reshape(ETOT // 128, 128)
    dst2d = jnp.concatenate(
        [edge_index[1], jnp.full((pad_e,), NPAD - 1, jnp.int32)]
    ).reshape(ETOT // 128, 128)
    batch2d = jnp.concatenate(
        [batch, jnp.full((NPAD - N,), 255, jnp.int32)]
    ).reshape(800, 128)
    zeros6400 = jnp.zeros((6400, 16), f32)
    z32 = jnp.zeros((256, 32), f32)
    ones1024 = jnp.ones((1024, 16), f32)
    b1r = b1.reshape(1, 16)
    b2r = b2.reshape(1, 32)
    bfr = bf.reshape(1, 3)

    deg2 = _p1_deg(dst2d, ones1024, zeros6400)
    dinv, g1 = _t1(xp, W1p, deg2[0], deg2[1])
    agg1 = _p2_agg1(src2d, dst2d, g1, zeros6400)
    g2a, g2b = _t2(agg1[0], agg1[1], g1, dinv, b1r, W2)
    agg2 = _p3_agg2(src2d, dst2d, g2a, g2b, zeros6400)
    p = _t3(agg2[0], agg2[1], g2a, g2b, dinv, b2r)
    ps, cs = _p4_pool(p, batch2d, z32, zeros6400, ones1024)
    return _t4(ps, cs, Wf, bfr)


# speedup vs baseline: 40.3969x; 1.8251x over previous
import functools

import jax
import jax.numpy as jnp
from jax import lax
from jax.experimental import pallas as pl
from jax.experimental.pallas import tpu as pltpu
from jax.experimental.pallas import tpu_sc as plsc

N = 100000
NPAD = 100352
ACC_ROWS = 100336
ASUB = ACC_ROWS // 16
PADID = ACC_ROWS - 1
E = 3200000
G = 128
NC, NS = 2, 16
NW = NC * NS
CH = 4
ECH = E // (CH * 128)
NCH1 = 196
NCH1_LAST = ECH - (NW - 1) * NCH1
NCH2 = 392
NCH2_LAST = ECH - (NS - 1) * NCH2

_MESH = dict(core_axis_name="c", subcore_axis_name="s", num_cores=NC,
             num_subcores=NS)


def _p1_deg(dst2d, ones512, zeros6400):
    @functools.partial(
        pl.kernel,
        out_type=jax.ShapeDtypeStruct((NC, ACC_ROWS, 16), jnp.float32),
        mesh=plsc.VectorSubcoreMesh(**_MESH),
        compiler_params=pltpu.CompilerParams(use_tc_tiling_on_sc=False),
        scratch_types=[
            pltpu.VMEM((2, CH * 128), jnp.int32),
            pltpu.VMEM((CH * 128, 16), jnp.float32),
            pltpu.VMEM_SHARED((ACC_ROWS, 16), jnp.float32),
            pltpu.SemaphoreType.DMA,
            pltpu.SemaphoreType.DMA,
        ],
    )
    def k(dst_hbm, ones_hbm, z_hbm, out_hbm, didx, obuf, acc, semi, sems):
        c = lax.axis_index("c")
        s = lax.axis_index("s")
        t = c * NS + s
        pltpu.sync_copy(z_hbm.at[pl.ds(0, ASUB)], acc.at[pl.ds(s * ASUB, ASUB)])
        pltpu.sync_copy(ones_hbm, obuf)
        plsc.subcore_barrier()
        base = t * NCH1
        nch = jnp.where(t == NW - 1, NCH1_LAST, NCH1)
        pltpu.async_copy(dst_hbm.at[base], didx.at[0], semi)

        def pair(g, carry):
            for b in (0, 1):
                j = 2 * g + b
                pltpu.make_async_copy(dst_hbm.at[0], didx.at[b],
                                      semi).wait()

                @pl.when(j >= 1)
                def _():
                    pltpu.make_async_copy(ones_hbm, obuf, sems).wait()

                @pl.when(j + 1 < nch)
                def _():
                    pltpu.async_copy(dst_hbm.at[base + j + 1],
                                     didx.at[1 - b], semi)

                pltpu.async_copy(obuf, acc.at[didx.at[b]], sems, add=True)
            return carry

        lax.fori_loop(0, nch // 2, pair, 0)
        pltpu.make_async_copy(ones_hbm, obuf, sems).wait()
        plsc.subcore_barrier()
        pltpu.sync_copy(acc.at[pl.ds(s * ASUB, ASUB)],
                        out_hbm.at[c, pl.ds(s * ASUB, ASUB)])

    return k(dst2d, ones512, zeros6400)


def _agg_pass(src2d, dst2d, ga, gb, zeros6400, layer1):
    @functools.partial(
        pl.kernel,
        out_type=jax.ShapeDtypeStruct((NC, ACC_ROWS, 16), jnp.float32),
        mesh=plsc.VectorSubcoreMesh(**_MESH),
        compiler_params=pltpu.CompilerParams(use_tc_tiling_on_sc=False),
        scratch_types=[
            pltpu.VMEM((2, CH * 128), jnp.int32),
            pltpu.VMEM((2, CH * 128), jnp.int32),
            pltpu.VMEM((2, CH * 128, 16), jnp.float32),
            pltpu.VMEM_SHARED((ACC_ROWS, 16), jnp.float32),
            pltpu.SemaphoreType.DMA,
            pltpu.SemaphoreType.DMA,
            pltpu.SemaphoreType.DMA,
        ],
    )
    def k(src_hbm, dst_hbm, ga_hbm, gb_hbm, z_hbm, out_hbm,
          sidx, didx, gbuf, acc, semi, semg, sems):
        c = lax.axis_index("c")
        s = lax.axis_index("s")
        pltpu.sync_copy(z_hbm.at[pl.ds(0, ASUB)], acc.at[pl.ds(s * ASUB, ASUB)])
        plsc.subcore_barrier()
        if layer1:
            t = c * NS + s
            base = t * NCH1
            nch = jnp.where(t == NW - 1, NCH1_LAST, NCH1)
        else:
            base = s * NCH2
            nch = jnp.where(s == NS - 1, NCH2_LAST, NCH2)

        def run(g_hbm):
            pltpu.async_copy(src_hbm.at[base], sidx.at[0], semi)
            pltpu.async_copy(dst_hbm.at[base], didx.at[0], semi)

            def pair(g, carry):
                for b in (0, 1):
                    j = 2 * g + b
                    pltpu.make_async_copy(src_hbm.at[0],
                                          sidx.at[b], semi).wait()
                    pltpu.make_async_copy(dst_hbm.at[0],
                                          didx.at[b], semi).wait()
                    gd = pltpu.async_copy(g_hbm.at[sidx.at[b]], gbuf.at[b],
                                          semg)
                    gd.wait()

                    @pl.when(j >= 1)
                    def _():
                        pltpu.make_async_copy(g_hbm.at[pl.ds(0, CH * 128)],
                                              gbuf.at[1 - b], sems).wait()

                    @pl.when(j + 1 < nch)
                    def _():
                        pltpu.async_copy(src_hbm.at[base + j + 1],
                                         sidx.at[1 - b], semi)
                        pltpu.async_copy(dst_hbm.at[base + j + 1],
                                         didx.at[1 - b], semi)

                    pltpu.async_copy(gbuf.at[b], acc.at[didx.at[b]], sems,
                                     add=True)
                return carry

            lax.fori_loop(0, nch // 2, pair, 0)
            pltpu.make_async_copy(g_hbm.at[pl.ds(0, CH * 128)], gbuf.at[1],
                                  sems).wait()

        if layer1:
            run(ga_hbm)
        else:
            @pl.when(c == 0)
            def _():
                run(ga_hbm)

            @pl.when(c == 1)
            def _():
                run(gb_hbm)

        plsc.subcore_barrier()
        pltpu.sync_copy(acc.at[pl.ds(s * ASUB, ASUB)],
                        out_hbm.at[c, pl.ds(s * ASUB, ASUB)])

    return k(src2d, dst2d, ga, gb, zeros6400)


def _p4_pool(p, batch2d, z32, zeros6400, ones512):
    @functools.partial(
        pl.kernel,
        out_type=(jax.ShapeDtypeStruct((NC, 256, 32), jnp.float32),
                  jax.ShapeDtypeStruct((NC, 256, 16), jnp.float32)),
        mesh=plsc.VectorSubcoreMesh(**_MESH),
        compiler_params=pltpu.CompilerParams(use_tc_tiling_on_sc=False),
        scratch_types=[
            pltpu.VMEM((128, 32), jnp.float32),
            pltpu.VMEM((28, 128), jnp.int32),
            pltpu.VMEM((128, 16), jnp.float32),
            pltpu.VMEM_SHARED((256, 32), jnp.float32),
            pltpu.VMEM_SHARED((256, 16), jnp.float32),
        ],
    )
    def k(p_hbm, b_hbm, z32_hbm, z16_hbm, ones_hbm, outs_hbm, outc_hbm,
          pbuf, bidx, onesv, pacc, cacc):
        c = lax.axis_index("c")
        s = lax.axis_index("s")
        t = c * NS + s

        @pl.when(s == 0)
        def _():
            pltpu.sync_copy(z32_hbm, pacc)
            pltpu.sync_copy(z16_hbm.at[pl.ds(0, 256)], cacc)

        pltpu.sync_copy(ones_hbm.at[pl.ds(0, 128)], onesv)
        plsc.subcore_barrier()

        @pl.when(t < 28)
        def _():
            pltpu.sync_copy(b_hbm.at[pl.ds(t * 28, 28)], bidx)
            for r in range(28):
                pltpu.sync_copy(p_hbm.at[pl.ds(t * 3584 + r * 128, 128)],
                                pbuf)
                pltpu.sync_copy(pbuf, pacc.at[bidx.at[r]], add=True)
                pltpu.sync_copy(onesv, cacc.at[bidx.at[r]], add=True)

        plsc.subcore_barrier()

        @pl.when(s == 0)
        def _():
            pltpu.sync_copy(pacc, outs_hbm.at[c])
            pltpu.sync_copy(cacc, outc_hbm.at[c])

    return k(p, batch2d, z32, zeros6400, ones512)


def _t1(xp, W1p, d0, d1):
    def body(x_ref, w_ref, d0_ref, d1_ref, dinv_ref, g1_ref):
        deg = d0_ref[...][:, 0:1] + d1_ref[...][:, 0:1] + 1.0
        dinv = lax.rsqrt(deg)
        h = jnp.dot(x_ref[...], w_ref[...], preferred_element_type=jnp.float32)
        dinv_ref[...] = dinv
        g1_ref[...] = h * dinv

    return pl.pallas_call(
        body,
        grid=(NPAD // 2048,),
        in_specs=[
            pl.BlockSpec((2048, 8), lambda j: (j, 0)),
            pl.BlockSpec((8, 16), lambda j: (0, 0)),
            pl.BlockSpec((2048, 16), lambda j: (j, 0)),
            pl.BlockSpec((2048, 16), lambda j: (j, 0)),
        ],
        out_specs=[
            pl.BlockSpec((2048, 1), lambda j: (j, 0)),
            pl.BlockSpec((2048, 16), lambda j: (j, 0)),
        ],
        out_shape=[
            jax.ShapeDtypeStruct((NPAD, 1), jnp.float32),
            jax.ShapeDtypeStruct((NPAD, 16), jnp.float32),
        ],
    )(xp, W1p, d0, d1)


def _t2(a0, a1, g1, dinv, b1r, W2):
    def body(a0_ref, a1_ref, g1_ref, d_ref, b_ref, w_ref, ga_ref, gb_ref):
        dinv = d_ref[...]
        o1 = jnp.maximum((a0_ref[...] + a1_ref[...] + g1_ref[...]) * dinv
                         + b_ref[...], 0.0)
        h2 = jnp.dot(o1, w_ref[...], preferred_element_type=jnp.float32)
        g2 = h2 * dinv
        ga_ref[...] = g2[:, :16]
        gb_ref[...] = g2[:, 16:]

    return pl.pallas_call(
        body,
        grid=(NPAD // 2048,),
        in_specs=[
            pl.BlockSpec((2048, 16), lambda j: (j, 0)),
            pl.BlockSpec((2048, 16), lambda j: (j, 0)),
            pl.BlockSpec((2048, 16), lambda j: (j, 0)),
            pl.BlockSpec((2048, 1), lambda j: (j, 0)),
            pl.BlockSpec((1, 16), lambda j: (0, 0)),
            pl.BlockSpec((16, 32), lambda j: (0, 0)),
        ],
        out_specs=[
            pl.BlockSpec((2048, 16), lambda j: (j, 0)),
            pl.BlockSpec((2048, 16), lambda j: (j, 0)),
        ],
        out_shape=[
            jax.ShapeDtypeStruct((NPAD, 16), jnp.float32),
            jax.ShapeDtypeStruct((NPAD, 16), jnp.float32),
        ],
    )(a0, a1, g1, dinv, b1r, W2)


def _t3(a2a, a2b, g2a, g2b, dinv, b2r):
    def body(aa_ref, ab_ref, ga_ref, gb_ref, d_ref, b_ref, p_ref):
        dinv = d_ref[...]
        b = b_ref[...]
        oa = jnp.maximum((aa_ref[...] + ga_ref[...]) * dinv + b[:, :16], 0.0)
        ob = jnp.maximum((ab_ref[...] + gb_ref[...]) * dinv + b[:, 16:], 0.0)
        p_ref[...] = jnp.concatenate([oa, ob], axis=1)

    return pl.pallas_call(
        body,
        grid=(NPAD // 2048,),
        in_specs=[
            pl.BlockSpec((2048, 16), lambda j: (j, 0)),
            pl.BlockSpec((2048, 16), lambda j: (j, 0)),
            pl.BlockSpec((2048, 16), lambda j: (j, 0)),
            pl.BlockSpec((2048, 16), lambda j: (j, 0)),
            pl.BlockSpec((2048, 1), lambda j: (j, 0)),
            pl.BlockSpec((1, 32), lambda j: (0, 0)),
        ],
        out_specs=pl.BlockSpec((2048, 32), lambda j: (j, 0)),
        out_shape=jax.ShapeDtypeStruct((NPAD, 32), jnp.float32),
    )(a2a, a2b, g2a, g2b, dinv, b2r)


def _t4(ps, cs, Wf, bfr):
    def body(ps_ref, cs_ref, wf_ref, bf_ref, out_ref):
        ssum = ps_ref[0, :G, :] + ps_ref[1, :G, :]
        cnt = cs_ref[0, :G, 0:1] + cs_ref[1, :G, 0:1]
        pooled = ssum / jnp.maximum(cnt, 1.0)
        out_ref[...] = jnp.dot(pooled, wf_ref[...],
                               preferred_element_type=jnp.float32) + bf_ref[...]

    return pl.pallas_call(
        body,
        out_shape=jax.ShapeDtypeStruct((G, 3), jnp.float32),
    )(ps, cs, Wf, bfr)


def kernel(x, edge_index, batch, W1, b1, W2, b2, Wf, bf):
    f32 = jnp.float32
    xp = jnp.zeros((NPAD, 8), f32).at[:N, :5].set(x)
    W1p = jnp.zeros((8, 16), f32).at[:5, :].set(W1)
    src2d = edge_index[0].reshape(ECH, CH * 128)
    dst2d = edge_index[1].reshape(ECH, CH * 128)
    batch2d = jnp.concatenate(
        [batch, jnp.full((NPAD - N,), 255, jnp.int32)]
    ).reshape(NPAD // 128, 128)
    zeros6400 = jnp.zeros((6400, 16), f32)
    z32 = jnp.zeros((256, 32), f32)
    ones512 = jnp.ones((CH * 128, 16), f32)
    b1r = b1.reshape(1, 16)
    b2r = b2.reshape(1, 32)
    bfr = bf.reshape(1, 3)

    deg2 = _p1_deg(dst2d, ones512, zeros6400)
    dinv, g1 = _t1(xp, W1p, deg2[0], deg2[1])
    agg1 = _agg_pass(src2d, dst2d, g1, g1, zeros6400, True)
    g2a, g2b = _t2(agg1[0], agg1[1], g1, dinv, b1r, W2)
    agg2 = _agg_pass(src2d, dst2d, g2a, g2b, zeros6400, False)
    p = _t3(agg2[0], agg2[1], g2a, g2b, dinv, b2r)
    ps, cs = _p4_pool(p, batch2d, z32, zeros6400, ones512)
    return _t4(ps, cs, Wf, bfr)


# speedup vs baseline: 42.0555x; 1.0411x over previous
import functools

import jax
import jax.numpy as jnp
from jax import lax
from jax.experimental import pallas as pl
from jax.experimental.pallas import tpu as pltpu
from jax.experimental.pallas import tpu_sc as plsc

N = 100000
NPAD = 100352
ACC_ROWS = 100336
ASUB = ACC_ROWS // 16
PADID = ACC_ROWS - 1
E = 3200000
G = 128
NC, NS = 2, 16
NW = NC * NS
CH = 4
ECH = E // (CH * 128)
NCH1 = 196
NCH1_LAST = ECH - (NW - 1) * NCH1
NCH2 = 392
NCH2_LAST = ECH - (NS - 1) * NCH2

_MESH = dict(core_axis_name="c", subcore_axis_name="s", num_cores=NC,
             num_subcores=NS)


def _p1_deg(dst2d, ones512, zeros6400):
    @functools.partial(
        pl.kernel,
        out_type=jax.ShapeDtypeStruct((NC, ACC_ROWS, 16), jnp.float32),
        mesh=plsc.VectorSubcoreMesh(**_MESH),
        compiler_params=pltpu.CompilerParams(use_tc_tiling_on_sc=False),
        scratch_types=[
            pltpu.VMEM((2, CH * 128), jnp.int32),
            pltpu.VMEM((CH * 128, 16), jnp.float32),
            pltpu.VMEM_SHARED((ACC_ROWS, 16), jnp.float32),
            pltpu.SemaphoreType.DMA,
            pltpu.SemaphoreType.DMA,
        ],
    )
    def k(dst_hbm, ones_hbm, z_hbm, out_hbm, didx, obuf, acc, semi, sems):
        c = lax.axis_index("c")
        s = lax.axis_index("s")
        t = c * NS + s
        pltpu.sync_copy(z_hbm.at[pl.ds(0, ASUB)], acc.at[pl.ds(s * ASUB, ASUB)])
        pltpu.sync_copy(ones_hbm, obuf)
        plsc.subcore_barrier()
        base = t * NCH1
        nch = jnp.where(t == NW - 1, NCH1_LAST, NCH1)
        pltpu.async_copy(dst_hbm.at[base], didx.at[0], semi)

        def pair(g, carry):
            for b in (0, 1):
                j = 2 * g + b
                pltpu.make_async_copy(dst_hbm.at[0], didx.at[b],
                                      semi).wait()

                @pl.when(j >= 1)
                def _():
                    pltpu.make_async_copy(ones_hbm, obuf, sems).wait()

                @pl.when(j + 1 < nch)
                def _():
                    pltpu.async_copy(dst_hbm.at[base + j + 1],
                                     didx.at[1 - b], semi)

                pltpu.async_copy(obuf, acc.at[didx.at[b]], sems, add=True)
            return carry

        lax.fori_loop(0, nch // 2, pair, 0)
        pltpu.make_async_copy(ones_hbm, obuf, sems).wait()
        plsc.subcore_barrier()
        pltpu.sync_copy(acc.at[pl.ds(s * ASUB, ASUB)],
                        out_hbm.at[c, pl.ds(s * ASUB, ASUB)])

    return k(dst2d, ones512, zeros6400)


def _agg_pass(src2d, dst2d, ga, gb, zeros6400, layer1):
    @functools.partial(
        pl.kernel,
        out_type=jax.ShapeDtypeStruct((NC, ACC_ROWS, 16), jnp.float32),
        mesh=plsc.VectorSubcoreMesh(**_MESH),
        compiler_params=pltpu.CompilerParams(use_tc_tiling_on_sc=False),
        scratch_types=[
            pltpu.VMEM((2, CH * 128), jnp.int32),
            pltpu.VMEM((2, CH * 128), jnp.int32),
            pltpu.VMEM((2, CH * 128, 16), jnp.float32),
            pltpu.VMEM_SHARED((ACC_ROWS, 16), jnp.float32),
            pltpu.SemaphoreType.DMA,
            pltpu.SemaphoreType.DMA,
            pltpu.SemaphoreType.DMA,
        ],
    )
    def k(src_hbm, dst_hbm, ga_hbm, gb_hbm, z_hbm, out_hbm,
          sidx, didx, gbuf, acc, semi, semg, sems):
        c = lax.axis_index("c")
        s = lax.axis_index("s")
        pltpu.sync_copy(z_hbm.at[pl.ds(0, ASUB)], acc.at[pl.ds(s * ASUB, ASUB)])
        plsc.subcore_barrier()
        if layer1:
            t = c * NS + s
            base = t * NCH1
            nch = jnp.where(t == NW - 1, NCH1_LAST, NCH1)
        else:
            base = s * NCH2
            nch = jnp.where(s == NS - 1, NCH2_LAST, NCH2)

        def run(g_hbm):
            pltpu.async_copy(src_hbm.at[base], sidx.at[0], semi)
            pltpu.async_copy(dst_hbm.at[base], didx.at[0], semi)

            def pair(g, carry):
                for b in (0, 1):
                    j = 2 * g + b
                    pltpu.make_async_copy(src_hbm.at[0],
                                          sidx.at[b], semi).wait()
                    pltpu.make_async_copy(dst_hbm.at[0],
                                          didx.at[b], semi).wait()
                    gd = pltpu.async_copy(g_hbm.at[sidx.at[b]], gbuf.at[b],
                                          semg)
                    gd.wait()

                    @pl.when(j >= 1)
                    def _():
                        pltpu.make_async_copy(g_hbm.at[pl.ds(0, CH * 128)],
                                              gbuf.at[1 - b], sems).wait()

                    @pl.when(j + 1 < nch)
                    def _():
                        pltpu.async_copy(src_hbm.at[base + j + 1],
                                         sidx.at[1 - b], semi)
                        pltpu.async_copy(dst_hbm.at[base + j + 1],
                                         didx.at[1 - b], semi)

                    pltpu.async_copy(gbuf.at[b], acc.at[didx.at[b]], sems,
                                     add=True)
                return carry

            lax.fori_loop(0, nch // 2, pair, 0)
            pltpu.make_async_copy(g_hbm.at[pl.ds(0, CH * 128)], gbuf.at[1],
                                  sems).wait()

        if layer1:
            run(ga_hbm)
        else:
            @pl.when(c == 0)
            def _():
                run(ga_hbm)

            @pl.when(c == 1)
            def _():
                run(gb_hbm)

        plsc.subcore_barrier()
        pltpu.sync_copy(acc.at[pl.ds(s * ASUB, ASUB)],
                        out_hbm.at[c, pl.ds(s * ASUB, ASUB)])

    return k(src2d, dst2d, ga, gb, zeros6400)


def _t1(xp, W1p, d0, d1):
    def body(x_ref, w_ref, d0_ref, d1_ref, dinv_ref, g1_ref):
        deg = d0_ref[...][:, 0:1] + d1_ref[...][:, 0:1] + 1.0
        dinv = lax.rsqrt(deg)
        h = jnp.dot(x_ref[...], w_ref[...], preferred_element_type=jnp.float32)
        dinv_ref[...] = dinv
        g1_ref[...] = h * dinv

    return pl.pallas_call(
        body,
        grid=(NPAD // 2048,),
        in_specs=[
            pl.BlockSpec((2048, 8), lambda j: (j, 0)),
            pl.BlockSpec((8, 16), lambda j: (0, 0)),
            pl.BlockSpec((2048, 16), lambda j: (j, 0)),
            pl.BlockSpec((2048, 16), lambda j: (j, 0)),
        ],
        out_specs=[
            pl.BlockSpec((2048, 1), lambda j: (j, 0)),
            pl.BlockSpec((2048, 16), lambda j: (j, 0)),
        ],
        out_shape=[
            jax.ShapeDtypeStruct((NPAD, 1), jnp.float32),
            jax.ShapeDtypeStruct((NPAD, 16), jnp.float32),
        ],
    )(xp, W1p, d0, d1)


def _t2(a0, a1, g1, dinv, b1r, W2):
    def body(a0_ref, a1_ref, g1_ref, d_ref, b_ref, w_ref, ga_ref, gb_ref):
        dinv = d_ref[...]
        o1 = jnp.maximum((a0_ref[...] + a1_ref[...] + g1_ref[...]) * dinv
                         + b_ref[...], 0.0)
        h2 = jnp.dot(o1, w_ref[...], preferred_element_type=jnp.float32)
        g2 = h2 * dinv
        ga_ref[...] = g2[:, :16]
        gb_ref[...] = g2[:, 16:]

    return pl.pallas_call(
        body,
        grid=(NPAD // 2048,),
        in_specs=[
            pl.BlockSpec((2048, 16), lambda j: (j, 0)),
            pl.BlockSpec((2048, 16), lambda j: (j, 0)),
            pl.BlockSpec((2048, 16), lambda j: (j, 0)),
            pl.BlockSpec((2048, 1), lambda j: (j, 0)),
            pl.BlockSpec((1, 16), lambda j: (0, 0)),
            pl.BlockSpec((16, 32), lambda j: (0, 0)),
        ],
        out_specs=[
            pl.BlockSpec((2048, 16), lambda j: (j, 0)),
            pl.BlockSpec((2048, 16), lambda j: (j, 0)),
        ],
        out_shape=[
            jax.ShapeDtypeStruct((NPAD, 16), jnp.float32),
            jax.ShapeDtypeStruct((NPAD, 16), jnp.float32),
        ],
    )(a0, a1, g1, dinv, b1r, W2)


def _t3(a2a, a2b, g2a, g2b, dinv, b2r, batch3d, Wf, bfr):
    nblk = NPAD // 2048

    def body(aa_ref, ab_ref, ga_ref, gb_ref, d_ref, b_ref, bat_ref, wf_ref,
             bf_ref, out_ref, sacc, cacc):
        j = pl.program_id(0)

        @pl.when(j == 0)
        def _():
            sacc[...] = jnp.zeros_like(sacc)
            cacc[...] = jnp.zeros_like(cacc)

        dinv = d_ref[...]
        b = b_ref[...]
        oa = jnp.maximum((aa_ref[...] + ga_ref[...]) * dinv + b[:, :16], 0.0)
        ob = jnp.maximum((ab_ref[...] + gb_ref[...]) * dinv + b[:, 16:], 0.0)
        pblk = jnp.concatenate([oa, ob], axis=1)
        bat = jnp.broadcast_to(bat_ref[0], (G, 2048))
        m = (jax.lax.broadcasted_iota(jnp.int32, (G, 2048), 0)
             == bat).astype(jnp.float32)
        sacc[...] += jnp.dot(m, pblk, preferred_element_type=jnp.float32)
        cacc[...] += jnp.sum(m, axis=1, keepdims=True)

        @pl.when(j == nblk - 1)
        def _():
            pooled = sacc[...] / jnp.maximum(cacc[...], 1.0)
            out_ref[...] = jnp.dot(pooled, wf_ref[...],
                                   preferred_element_type=jnp.float32) \
                           + bf_ref[...]

    return pl.pallas_call(
        body,
        grid=(nblk,),
        in_specs=[
            pl.BlockSpec((2048, 16), lambda j: (j, 0)),
            pl.BlockSpec((2048, 16), lambda j: (j, 0)),
            pl.BlockSpec((2048, 16), lambda j: (j, 0)),
            pl.BlockSpec((2048, 16), lambda j: (j, 0)),
            pl.BlockSpec((2048, 1), lambda j: (j, 0)),
            pl.BlockSpec((1, 32), lambda j: (0, 0)),
            pl.BlockSpec((1, 1, 2048), lambda j: (j, 0, 0)),
            pl.BlockSpec((32, 3), lambda j: (0, 0)),
            pl.BlockSpec((1, 3), lambda j: (0, 0)),
        ],
        out_specs=pl.BlockSpec((G, 3), lambda j: (0, 0)),
        out_shape=jax.ShapeDtypeStruct((G, 3), jnp.float32),
        scratch_shapes=[
            pltpu.VMEM((G, 32), jnp.float32),
            pltpu.VMEM((G, 1), jnp.float32),
        ],
    )(a2a, a2b, g2a, g2b, dinv, b2r, batch3d, Wf, bfr)


def kernel(x, edge_index, batch, W1, b1, W2, b2, Wf, bf):
    f32 = jnp.float32
    xp = jnp.zeros((NPAD, 8), f32).at[:N, :5].set(x)
    W1p = jnp.zeros((8, 16), f32).at[:5, :].set(W1)
    src2d = edge_index[0].reshape(ECH, CH * 128)
    dst2d = edge_index[1].reshape(ECH, CH * 128)
    batch3d = jnp.concatenate(
        [batch, jnp.full((NPAD - N,), 255, jnp.int32)]
    ).reshape(NPAD // 2048, 1, 2048)
    zeros6400 = jnp.zeros((6400, 16), f32)
    ones512 = jnp.ones((CH * 128, 16), f32)
    b1r = b1.reshape(1, 16)
    b2r = b2.reshape(1, 32)
    bfr = bf.reshape(1, 3)

    deg2 = _p1_deg(dst2d, ones512, zeros6400)
    dinv, g1 = _t1(xp, W1p, deg2[0], deg2[1])
    agg1 = _agg_pass(src2d, dst2d, g1, g1, zeros6400, True)
    g2a, g2b = _t2(agg1[0], agg1[1], g1, dinv, b1r, W2)
    agg2 = _agg_pass(src2d, dst2d, g2a, g2b, zeros6400, False)
    return _t3(agg2[0], agg2[1], g2a, g2b, dinv, b2r, batch3d, Wf, bfr)


# speedup vs baseline: 49.2589x; 1.1713x over previous
import functools

import jax
import jax.numpy as jnp
from jax import lax
from jax.experimental import pallas as pl
from jax.experimental.pallas import tpu as pltpu
from jax.experimental.pallas import tpu_sc as plsc

N = 100000
NPAD = 100352
ACC_ROWS = 100336
ASUB = ACC_ROWS // 16
PADID = ACC_ROWS - 1
E = 3200000
G = 128
NC, NS = 2, 16
NW = NC * NS
CHE = 256
ECH = E // CHE
NCH1 = 392
NCH1_LAST = ECH - (NW - 1) * NCH1
NCH2 = 784
NCH2_LAST = ECH - (NS - 1) * NCH2

_MESH = dict(core_axis_name="c", subcore_axis_name="s", num_cores=NC,
             num_subcores=NS)


def _p1_deg(dst2d, ones256, zeros6400):
    @functools.partial(
        pl.kernel,
        out_type=jax.ShapeDtypeStruct((NC, ACC_ROWS, 16), jnp.float32),
        mesh=plsc.VectorSubcoreMesh(**_MESH),
        compiler_params=pltpu.CompilerParams(use_tc_tiling_on_sc=False),
        scratch_types=[
            pltpu.VMEM((4, CHE), jnp.int32),
            pltpu.VMEM((CHE, 16), jnp.float32),
            pltpu.VMEM_SHARED((ACC_ROWS, 16), jnp.float32),
            pltpu.SemaphoreType.DMA,
            pltpu.SemaphoreType.DMA,
        ],
    )
    def k(dst_hbm, ones_hbm, z_hbm, out_hbm, didx, obuf, acc, semi, sems):
        c = lax.axis_index("c")
        s = lax.axis_index("s")
        t = c * NS + s
        pltpu.sync_copy(z_hbm.at[pl.ds(0, ASUB)], acc.at[pl.ds(s * ASUB, ASUB)])
        pltpu.sync_copy(ones_hbm, obuf)
        plsc.subcore_barrier()
        base = t * NCH1
        nch = jnp.where(t == NW - 1, NCH1_LAST, NCH1)
        pltpu.async_copy(dst_hbm.at[base], didx.at[0], semi)
        pltpu.async_copy(dst_hbm.at[base + 1], didx.at[1], semi)

        def quad(g, carry):
            for b in range(4):
                j = 4 * g + b
                b2 = (b + 2) % 4
                pltpu.make_async_copy(dst_hbm.at[0], didx.at[b], semi).wait()

                @pl.when(j >= 2)
                def _():
                    pltpu.make_async_copy(ones_hbm, obuf, sems).wait()

                @pl.when(j + 2 < nch)
                def _():
                    pltpu.async_copy(dst_hbm.at[base + j + 2], didx.at[b2],
                                     semi)

                pltpu.async_copy(obuf, acc.at[didx.at[b]], sems, add=True)
            return carry

        lax.fori_loop(0, nch // 4, quad, 0)
        pltpu.make_async_copy(ones_hbm, obuf, sems).wait()
        pltpu.make_async_copy(ones_hbm, obuf, sems).wait()
        plsc.subcore_barrier()
        pltpu.sync_copy(acc.at[pl.ds(s * ASUB, ASUB)],
                        out_hbm.at[c, pl.ds(s * ASUB, ASUB)])

    return k(dst2d, ones256, zeros6400)


def _agg_pass(src2d, dst2d, ga, gb, zeros6400, layer1):
    @functools.partial(
        pl.kernel,
        out_type=jax.ShapeDtypeStruct((NC, ACC_ROWS, 16), jnp.float32),
        mesh=plsc.VectorSubcoreMesh(**_MESH),
        compiler_params=pltpu.CompilerParams(use_tc_tiling_on_sc=False),
        scratch_types=[
            pltpu.VMEM((4, CHE), jnp.int32),
            pltpu.VMEM((4, CHE), jnp.int32),
            pltpu.VMEM((4, CHE, 16), jnp.float32),
            pltpu.VMEM_SHARED((ACC_ROWS, 16), jnp.float32),
            pltpu.SemaphoreType.DMA,
            pltpu.SemaphoreType.DMA,
            pltpu.SemaphoreType.DMA,
        ],
    )
    def k(src_hbm, dst_hbm, ga_hbm, gb_hbm, z_hbm, out_hbm,
          sidx, didx, gbuf, acc, semi, semg, sems):
        c = lax.axis_index("c")
        s = lax.axis_index("s")
        pltpu.sync_copy(z_hbm.at[pl.ds(0, ASUB)], acc.at[pl.ds(s * ASUB, ASUB)])
        plsc.subcore_barrier()
        if layer1:
            t = c * NS + s
            base = t * NCH1
            nch = jnp.where(t == NW - 1, NCH1_LAST, NCH1)
        else:
            base = s * NCH2
            nch = jnp.where(s == NS - 1, NCH2_LAST, NCH2)

        def run(g_hbm):
            pltpu.async_copy(src_hbm.at[base], sidx.at[0], semi)
            pltpu.async_copy(dst_hbm.at[base], didx.at[0], semi)
            pltpu.async_copy(src_hbm.at[base + 1], sidx.at[1], semi)
            pltpu.async_copy(dst_hbm.at[base + 1], didx.at[1], semi)

            def quad(g, carry):
                for b in range(4):
                    j = 4 * g + b
                    b1 = (b + 3) % 4
                    b2 = (b + 2) % 4
                    pltpu.make_async_copy(src_hbm.at[0], sidx.at[b],
                                          semi).wait()
                    pltpu.make_async_copy(dst_hbm.at[0], didx.at[b],
                                          semi).wait()

                    @pl.when(j >= 2)
                    def _():
                        pltpu.make_async_copy(g_hbm.at[pl.ds(0, CHE)],
                                              gbuf.at[b2], sems).wait()

                    pltpu.async_copy(g_hbm.at[sidx.at[b]], gbuf.at[b], semg)

                    @pl.when(j + 2 < nch)
                    def _():
                        pltpu.async_copy(src_hbm.at[base + j + 2],
                                         sidx.at[b2], semi)
                        pltpu.async_copy(dst_hbm.at[base + j + 2],
                                         didx.at[b2], semi)

                    @pl.when(j >= 1)
                    def _():
                        pltpu.make_async_copy(g_hbm.at[pl.ds(0, CHE)],
                                              gbuf.at[b1], semg).wait()
                        pltpu.async_copy(gbuf.at[b1], acc.at[didx.at[b1]],
                                        sems, add=True)
                return carry

            lax.fori_loop(0, nch // 4, quad, 0)
            pltpu.make_async_copy(g_hbm.at[pl.ds(0, CHE)], gbuf.at[3],
                                  semg).wait()
            pltpu.async_copy(gbuf.at[3], acc.at[didx.at[3]], sems, add=True)
            pltpu.make_async_copy(g_hbm.at[pl.ds(0, CHE)], gbuf.at[2],
                                  sems).wait()
            pltpu.make_async_copy(g_hbm.at[pl.ds(0, CHE)], gbuf.at[3],
                                  sems).wait()

        if layer1:
            run(ga_hbm)
        else:
            @pl.when(c == 0)
            def _():
                run(ga_hbm)

            @pl.when(c == 1)
            def _():
                run(gb_hbm)

        plsc.subcore_barrier()
        pltpu.sync_copy(acc.at[pl.ds(s * ASUB, ASUB)],
                        out_hbm.at[c, pl.ds(s * ASUB, ASUB)])

    return k(src2d, dst2d, ga, gb, zeros6400)


def _t1(xp, W1p, d0, d1):
    def body(x_ref, w_ref, d0_ref, d1_ref, dinv_ref, g1_ref):
        deg = d0_ref[...][:, 0:1] + d1_ref[...][:, 0:1] + 1.0
        dinv = lax.rsqrt(deg)
        h = jnp.dot(x_ref[...], w_ref[...], preferred_element_type=jnp.float32)
        dinv_ref[...] = dinv
        g1_ref[...] = h * dinv

    return pl.pallas_call(
        body,
        grid=(NPAD // 2048,),
        in_specs=[
            pl.BlockSpec((2048, 8), lambda j: (j, 0)),
            pl.BlockSpec((8, 16), lambda j: (0, 0)),
            pl.BlockSpec((2048, 16), lambda j: (j, 0)),
            pl.BlockSpec((2048, 16), lambda j: (j, 0)),
        ],
        out_specs=[
            pl.BlockSpec((2048, 1), lambda j: (j, 0)),
            pl.BlockSpec((2048, 16), lambda j: (j, 0)),
        ],
        out_shape=[
            jax.ShapeDtypeStruct((NPAD, 1), jnp.float32),
            jax.ShapeDtypeStruct((NPAD, 16), jnp.float32),
        ],
    )(xp, W1p, d0, d1)


def _t2(a0, a1, g1, dinv, b1r, W2):
    def body(a0_ref, a1_ref, g1_ref, d_ref, b_ref, w_ref, ga_ref, gb_ref):
        dinv = d_ref[...]
        o1 = jnp.maximum((a0_ref[...] + a1_ref[...] + g1_ref[...]) * dinv
                         + b_ref[...], 0.0)
        h2 = jnp.dot(o1, w_ref[...], preferred_element_type=jnp.float32)
        g2 = h2 * dinv
        ga_ref[...] = g2[:, :16]
        gb_ref[...] = g2[:, 16:]

    return pl.pallas_call(
        body,
        grid=(NPAD // 2048,),
        in_specs=[
            pl.BlockSpec((2048, 16), lambda j: (j, 0)),
            pl.BlockSpec((2048, 16), lambda j: (j, 0)),
            pl.BlockSpec((2048, 16), lambda j: (j, 0)),
            pl.BlockSpec((2048, 1), lambda j: (j, 0)),
            pl.BlockSpec((1, 16), lambda j: (0, 0)),
            pl.BlockSpec((16, 32), lambda j: (0, 0)),
        ],
        out_specs=[
            pl.BlockSpec((2048, 16), lambda j: (j, 0)),
            pl.BlockSpec((2048, 16), lambda j: (j, 0)),
        ],
        out_shape=[
            jax.ShapeDtypeStruct((NPAD, 16), jnp.float32),
            jax.ShapeDtypeStruct((NPAD, 16), jnp.float32),
        ],
    )(a0, a1, g1, dinv, b1r, W2)


def _t3(a2a, a2b, g2a, g2b, dinv, b2r, batch3d, Wf, bfr):
    nblk = NPAD // 2048

    def body(aa_ref, ab_ref, ga_ref, gb_ref, d_ref, b_ref, bat_ref, wf_ref,
             bf_ref, out_ref, sacc, cacc):
        j = pl.program_id(0)

        @pl.when(j == 0)
        def _():
            sacc[...] = jnp.zeros_like(sacc)
            cacc[...] = jnp.zeros_like(cacc)

        dinv = d_ref[...]
        b = b_ref[...]
        oa = jnp.maximum((aa_ref[...] + ga_ref[...]) * dinv + b[:, :16], 0.0)
        ob = jnp.maximum((ab_ref[...] + gb_ref[...]) * dinv + b[:, 16:], 0.0)
        pblk = jnp.concatenate([oa, ob], axis=1)
        bat = jnp.broadcast_to(bat_ref[0], (G, 2048))
        m = (jax.lax.broadcasted_iota(jnp.int32, (G, 2048), 0)
             == bat).astype(jnp.float32)
        sacc[...] += jnp.dot(m, pblk, preferred_element_type=jnp.float32)
        cacc[...] += jnp.sum(m, axis=1, keepdims=True)

        @pl.when(j == nblk - 1)
        def _():
            pooled = sacc[...] / jnp.maximum(cacc[...], 1.0)
            out_ref[...] = jnp.dot(pooled, wf_ref[...],
                                   preferred_element_type=jnp.float32) \
                           + bf_ref[...]

    return pl.pallas_call(
        body,
        grid=(nblk,),
        in_specs=[
            pl.BlockSpec((2048, 16), lambda j: (j, 0)),
            pl.BlockSpec((2048, 16), lambda j: (j, 0)),
            pl.BlockSpec((2048, 16), lambda j: (j, 0)),
            pl.BlockSpec((2048, 16), lambda j: (j, 0)),
            pl.BlockSpec((2048, 1), lambda j: (j, 0)),
            pl.BlockSpec((1, 32), lambda j: (0, 0)),
            pl.BlockSpec((1, 1, 2048), lambda j: (j, 0, 0)),
            pl.BlockSpec((32, 3), lambda j: (0, 0)),
            pl.BlockSpec((1, 3), lambda j: (0, 0)),
        ],
        out_specs=pl.BlockSpec((G, 3), lambda j: (0, 0)),
        out_shape=jax.ShapeDtypeStruct((G, 3), jnp.float32),
        scratch_shapes=[
            pltpu.VMEM((G, 32), jnp.float32),
            pltpu.VMEM((G, 1), jnp.float32),
        ],
    )(a2a, a2b, g2a, g2b, dinv, b2r, batch3d, Wf, bfr)


def kernel(x, edge_index, batch, W1, b1, W2, b2, Wf, bf):
    f32 = jnp.float32
    xp = jnp.zeros((NPAD, 8), f32).at[:N, :5].set(x)
    W1p = jnp.zeros((8, 16), f32).at[:5, :].set(W1)
    src2d = edge_index[0].reshape(ECH, CHE)
    dst2d = edge_index[1].reshape(ECH, CHE)
    batch3d = jnp.concatenate(
        [batch, jnp.full((NPAD - N,), 255, jnp.int32)]
    ).reshape(NPAD // 2048, 1, 2048)
    zeros6400 = jnp.zeros((6400, 16), f32)
    ones256 = jnp.ones((CHE, 16), f32)
    b1r = b1.reshape(1, 16)
    b2r = b2.reshape(1, 32)
    bfr = bf.reshape(1, 3)

    deg2 = _p1_deg(dst2d, ones256, zeros6400)
    dinv, g1 = _t1(xp, W1p, deg2[0], deg2[1])
    agg1 = _agg_pass(src2d, dst2d, g1, g1, zeros6400, True)
    g2a, g2b = _t2(agg1[0], agg1[1], g1, dinv, b1r, W2)
    agg2 = _agg_pass(src2d, dst2d, g2a, g2b, zeros6400, False)
    return _t3(agg2[0], agg2[1], g2a, g2b, dinv, b2r, batch3d, Wf, bfr)
